# baseline (speedup 1.0000x reference)
import functools

import jax
import jax.numpy as jnp
from jax import lax
from jax.experimental import pallas as pl
from jax.experimental.pallas import tpu as pltpu
from jax.experimental.pallas import tpu_sc as plsc

_B = 128
_G1 = 16
_G2 = 32
_NC = 2
_NS = 16
_PAD_ROWS = 112


def _seg_sweep(x_hbm, src_hbm, dst_hbm, rs0, rd0, isrc2, idst2, rows,
               sems_i, sems_g, sems_s, acc, nbt, gsz):
    ng = nbt // gsz

    def stage(g, sg):
        pltpu.async_copy(src_hbm.at[pl.ds(rs0 + g * gsz, gsz)], isrc2[sg],
                         sems_i[sg])
        pltpu.async_copy(dst_hbm.at[pl.ds(rd0 + g * gsz, gsz)], idst2[sg],
                         sems_i[sg])

    def wait_stage(g, sg):
        pltpu.make_async_copy(src_hbm.at[pl.ds(rs0 + g * gsz, gsz)],
                              isrc2[sg], sems_i[sg]).wait()
        pltpu.make_async_copy(dst_hbm.at[pl.ds(rd0 + g * gsz, gsz)],
                              idst2[sg], sems_i[sg]).wait()

    stage(0, 0)
    if ng > 1:
        stage(1, 1)

    for g in range(ng):
        sg = g & 1
        wait_stage(g, sg)
        isrc, idst = isrc2[sg], idst2[sg]

        def gather(j, s):
            pltpu.async_copy(x_hbm.at[isrc.at[j]], rows[s], sems_g[s])

        def wait_gather(j, s):
            pltpu.make_async_copy(x_hbm.at[isrc.at[j]], rows[s],
                                  sems_g[s]).wait()

        def scat(j, s):
            pltpu.async_copy(rows[s], acc.at[idst.at[j]], sems_s[s],
                             add=True)

        def wait_scat(j, s):
            pltpu.make_async_copy(rows[s], acc.at[idst.at[j]],
                                  sems_s[s]).wait()

        gather(0, 0)

        def outer(i, carry):
            for s in range(2):
                j = 2 * i + s

                wait_gather(j, s)
                scat(j, s)

                @pl.when(j >= 1)
                def _():
                    wait_scat(j - 1, 1 - s)

                @pl.when(j + 1 < gsz)
                def _():
                    gather(j + 1, 1 - s)
            return carry

        lax.fori_loop(0, gsz // 2, outer, 0)
        wait_scat(gsz - 1, 1)

        if g + 2 < ng:
            stage(g + 2, sg)


def _stripe_zero(z_hbm, acc, sid, acc_n):
    zs = acc_n // _NS
    pltpu.sync_copy(z_hbm.at[pl.ds(sid * zs, zs)], acc.at[pl.ds(sid * zs, zs)])


def _stripe_out(acc, out_hbm, sid, n, row0):
    w1 = (n // (8 * _NS)) * 8
    rem = n - _NS * w1
    pltpu.sync_copy(acc.at[pl.ds(sid * w1, w1)],
                    out_hbm.at[pl.ds(row0 + sid * w1, w1)])
    if rem:
        @pl.when(sid == _NS - 1)
        def _():
            pltpu.sync_copy(acc.at[pl.ds(_NS * w1, rem)],
                            out_hbm.at[pl.ds(row0 + _NS * w1, rem)])


def _segsum_l1(x, src2d, dst2d, zeros):
    n, d = x.shape
    nbp = src2d.shape[0]
    nbt = nbp // (_NC * _NS)
    acc_n = n + _PAD_ROWS

    mesh = plsc.VectorSubcoreMesh(core_axis_name="c", subcore_axis_name="s")

    @functools.partial(
        pl.kernel,
        out_type=jax.ShapeDtypeStruct((_NC * n, d), jnp.float32),
        mesh=mesh,
        scratch_types=[
            pltpu.VMEM((_G1, _B), jnp.int32),
            pltpu.VMEM((_G1, _B), jnp.int32),
            pltpu.VMEM((_G1, _B), jnp.int32),
            pltpu.VMEM((_G1, _B), jnp.int32),
            pltpu.VMEM((_B, d), jnp.float32),
            pltpu.VMEM((_B, d), jnp.float32),
            pltpu.VMEM_SHARED((acc_n, d), jnp.float32),
            pltpu.SemaphoreType.DMA,
            pltpu.SemaphoreType.DMA,
            pltpu.SemaphoreType.DMA,
            pltpu.SemaphoreType.DMA,
            pltpu.SemaphoreType.DMA,
            pltpu.SemaphoreType.DMA,
        ],
    )
    def k(x_hbm, src_hbm, dst_hbm, z_hbm, out_hbm,
          isa, isb, ida, idb, rows0, rows1, acc, semi0, semi1, semg0, semg1,
          sems0, sems1):
        cid = lax.axis_index("c")
        sid = lax.axis_index("s")
        wid = sid * _NC + cid

        _stripe_zero(z_hbm, acc, sid, acc_n)
        plsc.subcore_barrier()

        r0 = wid * nbt
        _seg_sweep(x_hbm, src_hbm, dst_hbm, r0, r0, (isa, isb), (ida, idb),
                   (rows0, rows1), (semi0, semi1), (semg0, semg1),
                   (sems0, sems1), acc, nbt, _G1)
        plsc.subcore_barrier()

        _stripe_out(acc, out_hbm, sid, n, cid * n)

    return k(x, src2d, dst2d, zeros)


def _segsum_l2(yflat, srcq, dst2d, zeros, n, nchunk):
    d = yflat.shape[1]
    nbp = dst2d.shape[0]
    nbt = nbp // _NS
    acc_n = n + _PAD_ROWS
    npass = nchunk // _NC

    mesh = plsc.VectorSubcoreMesh(core_axis_name="c", subcore_axis_name="s")

    @functools.partial(
        pl.kernel,
        out_type=jax.ShapeDtypeStruct((nchunk * n, d), jnp.float32),
        mesh=mesh,
        scratch_types=[
            pltpu.VMEM((_G2, _B), jnp.int32),
            pltpu.VMEM((_G2, _B), jnp.int32),
            pltpu.VMEM((_G2, _B), jnp.int32),
            pltpu.VMEM((_G2, _B), jnp.int32),
            pltpu.VMEM((_B, d), jnp.float32),
            pltpu.VMEM((_B, d), jnp.float32),
            pltpu.VMEM_SHARED((acc_n, d), jnp.float32),
            pltpu.SemaphoreType.DMA,
            pltpu.SemaphoreType.DMA,
            pltpu.SemaphoreType.DMA,
            pltpu.SemaphoreType.DMA,
            pltpu.SemaphoreType.DMA,
            pltpu.SemaphoreType.DMA,
        ],
    )
    def k(y_hbm, src_hbm, dst_hbm, z_hbm, out_hbm,
          isa, isb, ida, idb, rows0, rows1, acc, semi0, semi1, semg0, semg1,
          sems0, sems1):
        cid = lax.axis_index("c")
        sid = lax.axis_index("s")
        r0 = sid * nbt

        for p in range(npass):
            chunk = p * _NC + cid

            _stripe_zero(z_hbm, acc, sid, acc_n)
            plsc.subcore_barrier()
            _seg_sweep(y_hbm, src_hbm, dst_hbm, chunk * nbp + r0, r0,
                       (isa, isb), (ida, idb), (rows0, rows1),
                       (semi0, semi1), (semg0, semg1), (sems0, sems1),
                       acc, nbt, _G2)
            plsc.subcore_barrier()
            _stripe_out(acc, out_hbm, sid, n, chunk * n)

    return k(yflat, srcq, dst2d, zeros)


def _tc1(p2, W1, b1):
    _, n, d = p2.shape
    nv, sv, nh = W1.shape
    bn = 1000

    def body(p_ref, w_ref, b_ref, y_ref):
        a = p_ref[0] + p_ref[1]
        for v in range(nv):
            acc = jnp.dot(a[:, v * sv:(v + 1) * sv], w_ref[v],
                          preferred_element_type=jnp.float32)
            y_ref[v] = jnp.maximum(acc + b_ref[v][None, :], 0.0)

    return pl.pallas_call(
        body,
        grid=(n // bn,),
        in_specs=[
            pl.BlockSpec((2, bn, d), lambda i: (0, i, 0)),
            pl.BlockSpec((nv, sv, nh), lambda i: (0, 0, 0)),
            pl.BlockSpec((nv, nh), lambda i: (0, 0)),
        ],
        out_specs=pl.BlockSpec((nv, bn, nh), lambda i: (0, i, 0)),
        out_shape=jax.ShapeDtypeStruct((nv, n, nh), jnp.float32),
    )(p2, W1, b1)


def _tc2(a2flat, W2, b2, n, nv):
    nh = W2.shape[1]
    bn = 1000
    nblk = n // bn

    def body(a_ref, w_ref, b_ref, o_ref):
        acc = jnp.dot(a_ref[...], w_ref[...],
                      preferred_element_type=jnp.float32)
        o_ref[...] = jnp.maximum(acc + b_ref[...], 0.0)

    return pl.pallas_call(
        body,
        grid=(nblk, nv),
        in_specs=[
            pl.BlockSpec((bn, W2.shape[0]), lambda i, v: (v * nblk + i, 0)),
            pl.BlockSpec(W2.shape, lambda i, v: (0, 0)),
            pl.BlockSpec((1, nh), lambda i, v: (0, 0)),
        ],
        out_specs=pl.BlockSpec((bn, nh), lambda i, v: (i, v)),
        out_shape=jax.ShapeDtypeStruct((n, nv * nh), jnp.float32),
    )(a2flat, W2, b2.reshape(1, nh))


def kernel(h, g, W1, b1, W2, b2):
    n, d = h.shape
    nv = W1.shape[0]
    src = g[0].astype(jnp.int32)
    dst = g[1].astype(jnp.int32)
    e = src.shape[0]

    rows_unit = _B * _NC * _NS * 8
    ep = ((e + rows_unit - 1) // rows_unit) * rows_unit
    npad = ep - e
    pad_ar = jnp.arange(npad, dtype=jnp.int32)
    src_p = jnp.concatenate([src, pad_ar % 1024])
    dst_p = jnp.concatenate([dst, n + pad_ar % _PAD_ROWS])
    nbp = ep // _B
    src2d = src_p.reshape(nbp, _B)
    dst2d = dst_p.reshape(nbp, _B)
    zeros = jnp.zeros((n + _PAD_ROWS, d), jnp.float32)

    p2 = _segsum_l1(h, src2d, dst2d, zeros)
    y4 = _tc1(p2.reshape(_NC, n, d), W1, b1)

    nh = W1.shape[2]
    offs = (jnp.arange(nv, dtype=jnp.int32) * n)[:, None, None]
    srcq = (src2d[None] + offs).reshape(nv * nbp, _B)
    a2 = _segsum_l2(y4.reshape(nv * n, nh), srcq, dst2d, zeros, n, nv)
    return _tc2(a2, W2, b2, n, nv)


# speedup vs baseline: 1.1776x; 1.1776x over previous
import functools

import jax
import jax.numpy as jnp
from jax import lax
from jax.experimental import pallas as pl
from jax.experimental.pallas import tpu as pltpu
from jax.experimental.pallas import tpu_sc as plsc

_B = 128
_G1 = 16
_G2 = 32
_NC = 2
_NS = 16
_PAD_ROWS = 112


def _seg_sweep(x_hbm, src_hbm, dst_hbm, rs0, rd0, isrc2, idst2, rows,
               sems_i, sems_g, sems_s, acc, nbt, gsz):
    ng = nbt // gsz

    def stage(g, sg):
        pltpu.async_copy(src_hbm.at[pl.ds(rs0 + g * gsz, gsz)], isrc2[sg],
                         sems_i[sg])
        pltpu.async_copy(dst_hbm.at[pl.ds(rd0 + g * gsz, gsz)], idst2[sg],
                         sems_i[sg])

    def wait_stage(g, sg):
        pltpu.make_async_copy(src_hbm.at[pl.ds(rs0 + g * gsz, gsz)],
                              isrc2[sg], sems_i[sg]).wait()
        pltpu.make_async_copy(dst_hbm.at[pl.ds(rd0 + g * gsz, gsz)],
                              idst2[sg], sems_i[sg]).wait()

    stage(0, 0)
    if ng > 1:
        stage(1, 1)

    for g in range(ng):
        sg = g & 1
        wait_stage(g, sg)
        isrc, idst = isrc2[sg], idst2[sg]

        def start(j, s):
            pltpu.async_copy(x_hbm.at[isrc.at[j]], rows[s], sems_g[s])

        def fin(j, s):
            pltpu.make_async_copy(x_hbm.at[isrc.at[j]], rows[s],
                                  sems_g[s]).wait()
            pltpu.sync_copy(rows[s], acc.at[idst.at[j]], add=True)

        start(0, 0)
        start(1, 1)

        def outer(i, carry):
            for s in range(2):
                j = 2 * i + s

                fin(j, s)

                @pl.when(j + 2 < gsz)
                def _():
                    start(j + 2, s)
            return carry

        lax.fori_loop(0, gsz // 2, outer, 0)

        if g + 2 < ng:
            stage(g + 2, sg)


def _stripe_zero(z_hbm, acc, sid, acc_n):
    zs = acc_n // _NS
    pltpu.sync_copy(z_hbm.at[pl.ds(sid * zs, zs)], acc.at[pl.ds(sid * zs, zs)])


def _stripe_out(acc, out_hbm, sid, n, row0):
    w1 = (n // (8 * _NS)) * 8
    rem = n - _NS * w1
    pltpu.sync_copy(acc.at[pl.ds(sid * w1, w1)],
                    out_hbm.at[pl.ds(row0 + sid * w1, w1)])
    if rem:
        @pl.when(sid == _NS - 1)
        def _():
            pltpu.sync_copy(acc.at[pl.ds(_NS * w1, rem)],
                            out_hbm.at[pl.ds(row0 + _NS * w1, rem)])


def _segsum_l1(x, src2d, dst2d, zeros):
    n, d = x.shape
    nbp = src2d.shape[0]
    nbt = nbp // (_NC * _NS)
    acc_n = n + _PAD_ROWS

    mesh = plsc.VectorSubcoreMesh(core_axis_name="c", subcore_axis_name="s")

    @functools.partial(
        pl.kernel,
        out_type=jax.ShapeDtypeStruct((_NC * n, d), jnp.float32),
        mesh=mesh,
        scratch_types=[
            pltpu.VMEM((_G1, _B), jnp.int32),
            pltpu.VMEM((_G1, _B), jnp.int32),
            pltpu.VMEM((_G1, _B), jnp.int32),
            pltpu.VMEM((_G1, _B), jnp.int32),
            pltpu.VMEM((_B, d), jnp.float32),
            pltpu.VMEM((_B, d), jnp.float32),
            pltpu.VMEM_SHARED((acc_n, d), jnp.float32),
            pltpu.SemaphoreType.DMA,
            pltpu.SemaphoreType.DMA,
            pltpu.SemaphoreType.DMA,
            pltpu.SemaphoreType.DMA,
            pltpu.SemaphoreType.DMA,
            pltpu.SemaphoreType.DMA,
        ],
    )
    def k(x_hbm, src_hbm, dst_hbm, z_hbm, out_hbm,
          isa, isb, ida, idb, rows0, rows1, acc, semi0, semi1, semg0, semg1,
          sems0, sems1):
        cid = lax.axis_index("c")
        sid = lax.axis_index("s")
        wid = sid * _NC + cid

        _stripe_zero(z_hbm, acc, sid, acc_n)
        plsc.subcore_barrier()

        r0 = wid * nbt
        _seg_sweep(x_hbm, src_hbm, dst_hbm, r0, r0, (isa, isb), (ida, idb),
                   (rows0, rows1), (semi0, semi1), (semg0, semg1),
                   (sems0, sems1), acc, nbt, _G1)
        plsc.subcore_barrier()

        _stripe_out(acc, out_hbm, sid, n, cid * n)

    return k(x, src2d, dst2d, zeros)


def _segsum_l2(yflat, srcq, dst2d, zeros, n, nchunk):
    d = yflat.shape[1]
    nbp = dst2d.shape[0]
    nbt = nbp // _NS
    acc_n = n + _PAD_ROWS
    npass = nchunk // _NC

    mesh = plsc.VectorSubcoreMesh(core_axis_name="c", subcore_axis_name="s")

    @functools.partial(
        pl.kernel,
        out_type=jax.ShapeDtypeStruct((nchunk * n, d), jnp.float32),
        mesh=mesh,
        scratch_types=[
            pltpu.VMEM((_G2, _B), jnp.int32),
            pltpu.VMEM((_G2, _B), jnp.int32),
            pltpu.VMEM((_G2, _B), jnp.int32),
            pltpu.VMEM((_G2, _B), jnp.int32),
            pltpu.VMEM((_B, d), jnp.float32),
            pltpu.VMEM((_B, d), jnp.float32),
            pltpu.VMEM_SHARED((acc_n, d), jnp.float32),
            pltpu.SemaphoreType.DMA,
            pltpu.SemaphoreType.DMA,
            pltpu.SemaphoreType.DMA,
            pltpu.SemaphoreType.DMA,
            pltpu.SemaphoreType.DMA,
            pltpu.SemaphoreType.DMA,
        ],
    )
    def k(y_hbm, src_hbm, dst_hbm, z_hbm, out_hbm,
          isa, isb, ida, idb, rows0, rows1, acc, semi0, semi1, semg0, semg1,
          sems0, sems1):
        cid = lax.axis_index("c")
        sid = lax.axis_index("s")
        r0 = sid * nbt

        for p in range(npass):
            chunk = p * _NC + cid

            _stripe_zero(z_hbm, acc, sid, acc_n)
            plsc.subcore_barrier()
            _seg_sweep(y_hbm, src_hbm, dst_hbm, chunk * nbp + r0, r0,
                       (isa, isb), (ida, idb), (rows0, rows1),
                       (semi0, semi1), (semg0, semg1), (sems0, sems1),
                       acc, nbt, _G2)
            plsc.subcore_barrier()
            _stripe_out(acc, out_hbm, sid, n, chunk * n)

    return k(yflat, srcq, dst2d, zeros)


def _tc1(p2, W1, b1):
    _, n, d = p2.shape
    nv, sv, nh = W1.shape
    bn = 1000

    def body(p_ref, w_ref, b_ref, y_ref):
        a = p_ref[0] + p_ref[1]
        for v in range(nv):
            acc = jnp.dot(a[:, v * sv:(v + 1) * sv], w_ref[v],
                          preferred_element_type=jnp.float32)
            y_ref[v] = jnp.maximum(acc + b_ref[v][None, :], 0.0)

    return pl.pallas_call(
        body,
        grid=(n // bn,),
        in_specs=[
            pl.BlockSpec((2, bn, d), lambda i: (0, i, 0)),
            pl.BlockSpec((nv, sv, nh), lambda i: (0, 0, 0)),
            pl.BlockSpec((nv, nh), lambda i: (0, 0)),
        ],
        out_specs=pl.BlockSpec((nv, bn, nh), lambda i: (0, i, 0)),
        out_shape=jax.ShapeDtypeStruct((nv, n, nh), jnp.float32),
    )(p2, W1, b1)


def _tc2(a2flat, W2, b2, n, nv):
    nh = W2.shape[1]
    bn = 1000
    nblk = n // bn

    def body(a_ref, w_ref, b_ref, o_ref):
        acc = jnp.dot(a_ref[...], w_ref[...],
                      preferred_element_type=jnp.float32)
        o_ref[...] = jnp.maximum(acc + b_ref[...], 0.0)

    return pl.pallas_call(
        body,
        grid=(nblk, nv),
        in_specs=[
            pl.BlockSpec((bn, W2.shape[0]), lambda i, v: (v * nblk + i, 0)),
            pl.BlockSpec(W2.shape, lambda i, v: (0, 0)),
            pl.BlockSpec((1, nh), lambda i, v: (0, 0)),
        ],
        out_specs=pl.BlockSpec((bn, nh), lambda i, v: (i, v)),
        out_shape=jax.ShapeDtypeStruct((n, nv * nh), jnp.float32),
    )(a2flat, W2, b2.reshape(1, nh))


def kernel(h, g, W1, b1, W2, b2):
    n, d = h.shape
    nv = W1.shape[0]
    src = g[0].astype(jnp.int32)
    dst = g[1].astype(jnp.int32)
    e = src.shape[0]

    rows_unit = _B * _NC * _NS * 8
    ep = ((e + rows_unit - 1) // rows_unit) * rows_unit
    npad = ep - e
    pad_ar = jnp.arange(npad, dtype=jnp.int32)
    src_p = jnp.concatenate([src, pad_ar % 1024])
    dst_p = jnp.concatenate([dst, n + pad_ar % _PAD_ROWS])
    nbp = ep // _B
    src2d = src_p.reshape(nbp, _B)
    dst2d = dst_p.reshape(nbp, _B)
    zeros = jnp.zeros((n + _PAD_ROWS, d), jnp.float32)

    p2 = _segsum_l1(h, src2d, dst2d, zeros)
    y4 = _tc1(p2.reshape(_NC, n, d), W1, b1)

    nh = W1.shape[2]
    offs = (jnp.arange(nv, dtype=jnp.int32) * n)[:, None, None]
    srcq = (src2d[None] + offs).reshape(nv * nbp, _B)
    a2 = _segsum_l2(y4.reshape(nv * n, nh), srcq, dst2d, zeros, n, nv)
    return _tc2(a2, W2, b2, n, nv)


# speedup vs baseline: 1.2094x; 1.0270x over previous
import functools

import jax
import jax.numpy as jnp
from jax import lax
from jax.experimental import pallas as pl
from jax.experimental.pallas import tpu as pltpu
from jax.experimental.pallas import tpu_sc as plsc

_B = 128
_G1 = 16
_G2 = 32
_NC = 2
_NS = 16
_PAD_ROWS = 112


def _seg_sweep(x_hbm, src_hbm, dst_hbm, rs0, rd0, isrc2, idst2, rows,
               sems_i, sems_g, sems_s, acc, nbt, gsz):
    ng = nbt // gsz

    def stage(g, sg):
        pltpu.async_copy(src_hbm.at[pl.ds(rs0 + g * gsz, gsz)], isrc2[sg],
                         sems_i[sg])
        pltpu.async_copy(dst_hbm.at[pl.ds(rd0 + g * gsz, gsz)], idst2[sg],
                         sems_i[sg])

    def wait_stage(g, sg):
        pltpu.make_async_copy(src_hbm.at[pl.ds(rs0 + g * gsz, gsz)],
                              isrc2[sg], sems_i[sg]).wait()
        pltpu.make_async_copy(dst_hbm.at[pl.ds(rd0 + g * gsz, gsz)],
                              idst2[sg], sems_i[sg]).wait()

    def start(j):
        g, r = divmod(j, gsz)
        pltpu.async_copy(x_hbm.at[isrc2[g & 1].at[r]], rows[j & 1],
                         sems_g[j & 1])

    def fin(j):
        g, r = divmod(j, gsz)
        pltpu.make_async_copy(x_hbm.at[isrc2[g & 1].at[r]], rows[j & 1],
                              sems_g[j & 1]).wait()
        pltpu.sync_copy(rows[j & 1], acc.at[idst2[g & 1].at[r]], add=True)

    def prime():
        stage(0, 0)
        if ng > 1:
            stage(1, 1)
        wait_stage(0, 0)
        start(0)
        start(1)

    def ring():
        for j in range(nbt):
            fin(j)
            g, r = divmod(j, gsz)
            if r == gsz - 1 and g + 2 < ng:
                stage(g + 2, g & 1)
            nxt = j + 2
            if nxt < nbt:
                if nxt % gsz == 0:
                    wait_stage(nxt // gsz, (nxt // gsz) & 1)
                start(nxt)

    return prime, ring


def _stripe_zero(z_hbm, acc, sid, acc_n, sem_z):
    zs = acc_n // _NS
    pltpu.async_copy(z_hbm.at[pl.ds(sid * zs, zs)],
                     acc.at[pl.ds(sid * zs, zs)], sem_z)


def _wait_stripe_zero(z_hbm, acc, sid, acc_n, sem_z):
    zs = acc_n // _NS
    pltpu.make_async_copy(z_hbm.at[pl.ds(sid * zs, zs)],
                          acc.at[pl.ds(sid * zs, zs)], sem_z).wait()


def _stripe_out(acc, out_hbm, sid, n, row0):
    w1 = (n // (8 * _NS)) * 8
    rem = n - _NS * w1
    pltpu.sync_copy(acc.at[pl.ds(sid * w1, w1)],
                    out_hbm.at[pl.ds(row0 + sid * w1, w1)])
    if rem:
        @pl.when(sid == _NS - 1)
        def _():
            pltpu.sync_copy(acc.at[pl.ds(_NS * w1, rem)],
                            out_hbm.at[pl.ds(row0 + _NS * w1, rem)])


def _segsum_l1(x, src2d, dst2d, zeros):
    n, d = x.shape
    nbp = src2d.shape[0]
    nbt = nbp // (_NC * _NS)
    acc_n = n + _PAD_ROWS

    mesh = plsc.VectorSubcoreMesh(core_axis_name="c", subcore_axis_name="s")

    @functools.partial(
        pl.kernel,
        out_type=jax.ShapeDtypeStruct((_NC * n, d), jnp.float32),
        mesh=mesh,
        scratch_types=[
            pltpu.VMEM((_G1, _B), jnp.int32),
            pltpu.VMEM((_G1, _B), jnp.int32),
            pltpu.VMEM((_G1, _B), jnp.int32),
            pltpu.VMEM((_G1, _B), jnp.int32),
            pltpu.VMEM((_B, d), jnp.float32),
            pltpu.VMEM((_B, d), jnp.float32),
            pltpu.VMEM_SHARED((acc_n, d), jnp.float32),
            pltpu.SemaphoreType.DMA,
            pltpu.SemaphoreType.DMA,
            pltpu.SemaphoreType.DMA,
            pltpu.SemaphoreType.DMA,
            pltpu.SemaphoreType.DMA,
            pltpu.SemaphoreType.DMA,
            pltpu.SemaphoreType.DMA,
        ],
    )
    def k(x_hbm, src_hbm, dst_hbm, z_hbm, out_hbm,
          isa, isb, ida, idb, rows0, rows1, acc, semi0, semi1, semg0, semg1,
          sems0, sems1, semz):
        cid = lax.axis_index("c")
        sid = lax.axis_index("s")
        wid = sid * _NC + cid

        r0 = wid * nbt
        prime, ring = _seg_sweep(x_hbm, src_hbm, dst_hbm, r0, r0,
                                 (isa, isb), (ida, idb), (rows0, rows1),
                                 (semi0, semi1), (semg0, semg1),
                                 (sems0, sems1), acc, nbt, _G1)
        _stripe_zero(z_hbm, acc, sid, acc_n, semz)
        prime()
        _wait_stripe_zero(z_hbm, acc, sid, acc_n, semz)
        plsc.subcore_barrier()
        ring()
        plsc.subcore_barrier()

        _stripe_out(acc, out_hbm, sid, n, cid * n)

    return k(x, src2d, dst2d, zeros)


def _segsum_l2(yflat, srcq, dst2d, zeros, n, nchunk):
    d = yflat.shape[1]
    nbp = dst2d.shape[0]
    nbt = nbp // _NS
    acc_n = n + _PAD_ROWS
    npass = nchunk // _NC

    mesh = plsc.VectorSubcoreMesh(core_axis_name="c", subcore_axis_name="s")

    @functools.partial(
        pl.kernel,
        out_type=jax.ShapeDtypeStruct((nchunk * n, d), jnp.float32),
        mesh=mesh,
        scratch_types=[
            pltpu.VMEM((_G2, _B), jnp.int32),
            pltpu.VMEM((_G2, _B), jnp.int32),
            pltpu.VMEM((_G2, _B), jnp.int32),
            pltpu.VMEM((_G2, _B), jnp.int32),
            pltpu.VMEM((_B, d), jnp.float32),
            pltpu.VMEM((_B, d), jnp.float32),
            pltpu.VMEM_SHARED((acc_n, d), jnp.float32),
            pltpu.SemaphoreType.DMA,
            pltpu.SemaphoreType.DMA,
            pltpu.SemaphoreType.DMA,
            pltpu.SemaphoreType.DMA,
            pltpu.SemaphoreType.DMA,
            pltpu.SemaphoreType.DMA,
            pltpu.SemaphoreType.DMA,
        ],
    )
    def k(y_hbm, src_hbm, dst_hbm, z_hbm, out_hbm,
          isa, isb, ida, idb, rows0, rows1, acc, semi0, semi1, semg0, semg1,
          sems0, sems1, semz):
        cid = lax.axis_index("c")
        sid = lax.axis_index("s")
        r0 = sid * nbt

        for p in range(npass):
            chunk = p * _NC + cid

            prime, ring = _seg_sweep(y_hbm, src_hbm, dst_hbm,
                                     chunk * nbp + r0, r0,
                                     (isa, isb), (ida, idb), (rows0, rows1),
                                     (semi0, semi1), (semg0, semg1),
                                     (sems0, sems1), acc, nbt, _G2)
            _stripe_zero(z_hbm, acc, sid, acc_n, semz)
            prime()
            _wait_stripe_zero(z_hbm, acc, sid, acc_n, semz)
            plsc.subcore_barrier()
            ring()
            plsc.subcore_barrier()
            _stripe_out(acc, out_hbm, sid, n, chunk * n)
            if p + 1 < npass:
                plsc.subcore_barrier()

    return k(yflat, srcq, dst2d, zeros)


def _tc1(p2, W1, b1):
    _, n, d = p2.shape
    nv, sv, nh = W1.shape
    bn = 1000

    def body(p_ref, w_ref, b_ref, y_ref):
        a = p_ref[0] + p_ref[1]
        for v in range(nv):
            acc = jnp.dot(a[:, v * sv:(v + 1) * sv], w_ref[v],
                          preferred_element_type=jnp.float32)
            y_ref[v] = jnp.maximum(acc + b_ref[v][None, :], 0.0)

    return pl.pallas_call(
        body,
        grid=(n // bn,),
        in_specs=[
            pl.BlockSpec((2, bn, d), lambda i: (0, i, 0)),
            pl.BlockSpec((nv, sv, nh), lambda i: (0, 0, 0)),
            pl.BlockSpec((nv, nh), lambda i: (0, 0)),
        ],
        out_specs=pl.BlockSpec((nv, bn, nh), lambda i: (0, i, 0)),
        out_shape=jax.ShapeDtypeStruct((nv, n, nh), jnp.float32),
    )(p2, W1, b1)


def _tc2(a2flat, W2, b2, n, nv):
    nh = W2.shape[1]
    bn = 1000
    nblk = n // bn

    def body(a_ref, w_ref, b_ref, o_ref):
        acc = jnp.dot(a_ref[...], w_ref[...],
                      preferred_element_type=jnp.float32)
        o_ref[...] = jnp.maximum(acc + b_ref[...], 0.0)

    return pl.pallas_call(
        body,
        grid=(nblk, nv),
        in_specs=[
            pl.BlockSpec((bn, W2.shape[0]), lambda i, v: (v * nblk + i, 0)),
            pl.BlockSpec(W2.shape, lambda i, v: (0, 0)),
            pl.BlockSpec((1, nh), lambda i, v: (0, 0)),
        ],
        out_specs=pl.BlockSpec((bn, nh), lambda i, v: (i, v)),
        out_shape=jax.ShapeDtypeStruct((n, nv * nh), jnp.float32),
    )(a2flat, W2, b2.reshape(1, nh))


def kernel(h, g, W1, b1, W2, b2):
    n, d = h.shape
    nv = W1.shape[0]
    src = g[0].astype(jnp.int32)
    dst = g[1].astype(jnp.int32)
    e = src.shape[0]

    rows_unit = _B * _NC * _NS * 8
    ep = ((e + rows_unit - 1) // rows_unit) * rows_unit
    npad = ep - e
    pad_ar = jnp.arange(npad, dtype=jnp.int32)
    src_p = jnp.concatenate([src, pad_ar % 1024])
    dst_p = jnp.concatenate([dst, n + pad_ar % _PAD_ROWS])
    nbp = ep // _B
    src2d = src_p.reshape(nbp, _B)
    dst2d = dst_p.reshape(nbp, _B)
    zeros = jnp.zeros((n + _PAD_ROWS, d), jnp.float32)

    p2 = _segsum_l1(h, src2d, dst2d, zeros)
    y4 = _tc1(p2.reshape(_NC, n, d), W1, b1)

    nh = W1.shape[2]
    offs = (jnp.arange(nv, dtype=jnp.int32) * n)[:, None, None]
    srcq = (src2d[None] + offs).reshape(nv * nbp, _B)
    a2 = _segsum_l2(y4.reshape(nv * n, nh), srcq, dst2d, zeros, n, nv)
    return _tc2(a2, W2, b2, n, nv)
